# baseline (speedup 1.0000x reference)
import functools

import jax
import jax.numpy as jnp
from jax import lax
from jax.experimental import pallas as pl
from jax.experimental.pallas import tpu as pltpu
from jax.experimental.pallas import tpu_sc as plsc

D_MODEL = 1024
BATCH = 4
SEQ_LEN = 2048
OUT_LEN = SEQ_LEN + 1
NUM_WORKERS = 32
ROWS_PER_WORKER = BATCH * SEQ_LEN // NUM_WORKERS
WORKERS_PER_BATCH = NUM_WORKERS // BATCH
CHUNK = 32
NCHUNKS = ROWS_PER_WORKER // CHUNK
IPAD = 8


def _encode(tokens3, intensity_pad, seq_table, int_table):
    mesh = plsc.VectorSubcoreMesh(core_axis_name="c", subcore_axis_name="s")

    @functools.partial(
        pl.kernel,
        mesh=mesh,
        out_type=jax.ShapeDtypeStruct((BATCH, OUT_LEN, D_MODEL), jnp.float32),
        scratch_types=[
            pltpu.VMEM((NCHUNKS, CHUNK), jnp.int32),
            pltpu.VMEM((CHUNK, D_MODEL), jnp.float32),
            pltpu.VMEM((CHUNK, D_MODEL), jnp.float32),
            pltpu.VMEM((IPAD,), jnp.int32),
            pltpu.VMEM((IPAD, D_MODEL), jnp.float32),
            pltpu.SemaphoreType.DMA,
            pltpu.SemaphoreType.DMA,
            pltpu.SemaphoreType.DMA,
        ],
    )
    def k(tok_hbm, iid_hbm, seqt_hbm, intt_hbm, out_hbm,
          idx_v, buf0, buf1, iidx_v, irows_v, sem0, sem1, isem):
        wid = lax.axis_index("s") * 2 + lax.axis_index("c")
        b = wid // WORKERS_PER_BATCH
        part = wid % WORKERS_PER_BATCH
        dst_base = part * ROWS_PER_WORKER

        pltpu.sync_copy(tok_hbm.at[wid], idx_v)

        bufs = (buf0, buf1)
        sems = (sem0, sem1)
        copies = [None] * NCHUNKS
        copies[0] = pltpu.async_copy(seqt_hbm.at[idx_v.at[0]], buf0, sem0)
        for j in range(NCHUNKS):
            if j + 1 < NCHUNKS:
                copies[j + 1] = pltpu.async_copy(
                    seqt_hbm.at[idx_v.at[j + 1]], bufs[(j + 1) % 2],
                    sems[(j + 1) % 2])
            copies[j].wait()
            pltpu.sync_copy(
                bufs[j % 2],
                out_hbm.at[b, pl.ds(dst_base + j * CHUNK, CHUNK)])

        @pl.when(wid == 0)
        def _():
            pltpu.sync_copy(iid_hbm, iidx_v)
            pltpu.async_copy(intt_hbm.at[iidx_v], irows_v, isem).wait()
            for bb in range(BATCH):
                pltpu.sync_copy(
                    irows_v.at[pl.ds(bb, 1)],
                    out_hbm.at[bb, pl.ds(SEQ_LEN, 1)])

    return k(tokens3, intensity_pad, seq_table, int_table)


def kernel(sequence_tokens, intensity_ids, seq_table, int_table):
    tokens3 = sequence_tokens.reshape(
        NUM_WORKERS, NCHUNKS, CHUNK).astype(jnp.int32)
    intensity_pad = jnp.zeros((IPAD,), jnp.int32).at[:BATCH].set(
        intensity_ids.astype(jnp.int32))
    return _encode(tokens3, intensity_pad, seq_table, int_table)


# speedup vs baseline: 1.4759x; 1.4759x over previous
import functools

import jax
import jax.numpy as jnp
from jax import lax
from jax.experimental import pallas as pl
from jax.experimental.pallas import tpu as pltpu
from jax.experimental.pallas import tpu_sc as plsc

D_MODEL = 1024
BATCH = 4
SEQ_LEN = 2048
OUT_LEN = SEQ_LEN + 1
SEQ_VOCAB = 30
NUM_WORKERS = 32
ROWS_PER_WORKER = BATCH * SEQ_LEN // NUM_WORKERS
WORKERS_PER_BATCH = NUM_WORKERS // BATCH
CHUNK = 32
NCHUNKS = ROWS_PER_WORKER // CHUNK
LANES = 16
IPAD = 8


def _encode(tokens3, intensity_pad, seq_rep, int_table):
    mesh = plsc.VectorSubcoreMesh(core_axis_name="c", subcore_axis_name="s")

    @functools.partial(
        pl.kernel,
        mesh=mesh,
        out_type=jax.ShapeDtypeStruct((BATCH, OUT_LEN, D_MODEL), jnp.float32),
        scratch_types=[
            pltpu.VMEM((NCHUNKS, CHUNK), jnp.int32),
            pltpu.VMEM((CHUNK, D_MODEL), jnp.float32),
            pltpu.VMEM((CHUNK, D_MODEL), jnp.float32),
            pltpu.VMEM((IPAD,), jnp.int32),
            pltpu.VMEM((IPAD, D_MODEL), jnp.float32),
            pltpu.SemaphoreType.DMA,
            pltpu.SemaphoreType.DMA,
            pltpu.SemaphoreType.DMA,
        ],
    )
    def k(tok_hbm, iid_hbm, seqt_hbm, intt_hbm, out_hbm,
          idx_v, buf0, buf1, iidx_v, irows_v, sem0, sem1, isem):
        wid = lax.axis_index("s") * 2 + lax.axis_index("c")
        b = wid // WORKERS_PER_BATCH
        part = wid % WORKERS_PER_BATCH
        dst_base = part * ROWS_PER_WORKER

        pltpu.sync_copy(tok_hbm.at[wid], idx_v)

        off = jnp.full((LANES,), SEQ_VOCAB, jnp.int32) * wid
        for j in range(NCHUNKS):
            for kk in range(CHUNK // LANES):
                sl = pl.ds(kk * LANES, LANES)
                idx_v[j, sl] = idx_v[j, sl] + off

        bufs = (buf0, buf1)
        sems = (sem0, sem1)
        copies = [None] * NCHUNKS
        copies[0] = pltpu.async_copy(seqt_hbm.at[idx_v.at[0]], buf0, sem0)
        for j in range(NCHUNKS):
            if j + 1 < NCHUNKS:
                copies[j + 1] = pltpu.async_copy(
                    seqt_hbm.at[idx_v.at[j + 1]], bufs[(j + 1) % 2],
                    sems[(j + 1) % 2])
            copies[j].wait()
            pltpu.sync_copy(
                bufs[j % 2],
                out_hbm.at[b, pl.ds(dst_base + j * CHUNK, CHUNK)])

        @pl.when(wid == 0)
        def _():
            pltpu.sync_copy(iid_hbm, iidx_v)
            pltpu.async_copy(intt_hbm.at[iidx_v], irows_v, isem).wait()
            for bb in range(BATCH):
                pltpu.sync_copy(
                    irows_v.at[pl.ds(bb, 1)],
                    out_hbm.at[bb, pl.ds(SEQ_LEN, 1)])

    return k(tokens3, intensity_pad, seq_rep, int_table)


def kernel(sequence_tokens, intensity_ids, seq_table, int_table):
    tokens3 = sequence_tokens.reshape(
        NUM_WORKERS, NCHUNKS, CHUNK).astype(jnp.int32)
    intensity_pad = jnp.zeros((IPAD,), jnp.int32).at[:BATCH].set(
        intensity_ids.astype(jnp.int32))
    seq_rep = jnp.tile(seq_table, (NUM_WORKERS, 1))
    return _encode(tokens3, intensity_pad, seq_rep, int_table)


# speedup vs baseline: 2.8930x; 1.9602x over previous
import functools

import jax
import jax.numpy as jnp
from jax import lax
from jax.experimental import pallas as pl
from jax.experimental.pallas import tpu as pltpu
from jax.experimental.pallas import tpu_sc as plsc

D_MODEL = 1024
DBLK = D_MODEL // 128
BATCH = 4
SEQ_LEN = 2048
OUT_LEN = SEQ_LEN + 1
SEQ_VOCAB = 30
NUM_WORKERS = 32
ROWS_PER_WORKER = BATCH * SEQ_LEN // NUM_WORKERS
WORKERS_PER_BATCH = NUM_WORKERS // BATCH
CHUNK = 32
NCHUNKS = ROWS_PER_WORKER // CHUNK
LANES = 16
IPAD = 8


def _encode(tok_flat, intensity_pad, seq_rep, int_tab3):
    mesh = plsc.VectorSubcoreMesh(core_axis_name="c", subcore_axis_name="s")

    @functools.partial(
        pl.kernel,
        mesh=mesh,
        out_type=jax.ShapeDtypeStruct((OUT_LEN, DBLK, BATCH, 128),
                                      jnp.float32),
        scratch_types=[
            pltpu.VMEM((ROWS_PER_WORKER,), jnp.int32),
            pltpu.VMEM((CHUNK, DBLK, 128), jnp.float32),
            pltpu.VMEM((CHUNK, DBLK, 128), jnp.float32),
            pltpu.VMEM((IPAD,), jnp.int32),
            pltpu.VMEM((IPAD, DBLK, 128), jnp.float32),
            pltpu.SemaphoreType.DMA,
            pltpu.SemaphoreType.DMA,
            pltpu.SemaphoreType.DMA,
        ],
    )
    def k(tok_hbm, iid_hbm, seqt_hbm, intt_hbm, out_hbm,
          idx_v, buf0, buf1, iidx_v, irows_v, sem0, sem1, isem):
        wid = lax.axis_index("s") * 2 + lax.axis_index("c")
        b = wid // WORKERS_PER_BATCH
        part = wid % WORKERS_PER_BATCH
        dst_base = part * ROWS_PER_WORKER

        pltpu.sync_copy(tok_hbm.at[pl.ds(wid * ROWS_PER_WORKER,
                                         ROWS_PER_WORKER)], idx_v)

        off = jnp.full((LANES,), SEQ_VOCAB, jnp.int32) * wid
        for kk in range(ROWS_PER_WORKER // LANES):
            sl = pl.ds(kk * LANES, LANES)
            idx_v[sl] = idx_v[sl] + off

        bufs = (buf0, buf1)
        sems = (sem0, sem1)
        copies = [None] * NCHUNKS
        copies[0] = pltpu.async_copy(
            seqt_hbm.at[idx_v.at[pl.ds(0, CHUNK)]], buf0, sem0)
        for j in range(NCHUNKS):
            if j + 1 < NCHUNKS:
                copies[j + 1] = pltpu.async_copy(
                    seqt_hbm.at[idx_v.at[pl.ds((j + 1) * CHUNK, CHUNK)]],
                    bufs[(j + 1) % 2], sems[(j + 1) % 2])
            copies[j].wait()
            pltpu.sync_copy(
                bufs[j % 2],
                out_hbm.at[pl.ds(dst_base + j * CHUNK, CHUNK), :, b, :])

        @pl.when(wid == 0)
        def _():
            pltpu.sync_copy(iid_hbm, iidx_v)
            pltpu.async_copy(intt_hbm.at[iidx_v], irows_v, isem).wait()
            for bb in range(BATCH):
                pltpu.sync_copy(
                    irows_v.at[pl.ds(bb, 1)],
                    out_hbm.at[pl.ds(SEQ_LEN, 1), :, bb, :])

    return k(tok_flat, intensity_pad, seq_rep, int_tab3)


def kernel(sequence_tokens, intensity_ids, seq_table, int_table):
    tok_flat = sequence_tokens.reshape(BATCH * SEQ_LEN).astype(jnp.int32)
    intensity_pad = jnp.zeros((IPAD,), jnp.int32).at[:BATCH].set(
        intensity_ids.astype(jnp.int32))
    seq_rep = jnp.tile(seq_table, (NUM_WORKERS, 1)).reshape(
        NUM_WORKERS * SEQ_VOCAB, DBLK, 128)
    int_tab3 = int_table.reshape(-1, DBLK, 128)
    out4 = _encode(tok_flat, intensity_pad, seq_rep, int_tab3)
    return out4.transpose(2, 0, 1, 3).reshape(BATCH, OUT_LEN, D_MODEL)


# speedup vs baseline: 2.9782x; 1.0295x over previous
import functools

import jax
import jax.numpy as jnp
from jax import lax
from jax.experimental import pallas as pl
from jax.experimental.pallas import tpu as pltpu
from jax.experimental.pallas import tpu_sc as plsc

D_MODEL = 1024
DBLK = D_MODEL // 128
BATCH = 4
SEQ_LEN = 2048
OUT_LEN = SEQ_LEN + 1
SEQ_VOCAB = 30
NUM_WORKERS = 32
NREP = 16
ROWS_PER_WORKER = BATCH * SEQ_LEN // NUM_WORKERS
WORKERS_PER_BATCH = NUM_WORKERS // BATCH
TOKBLK = ROWS_PER_WORKER // 128
CHUNK = 32
NCHUNKS = ROWS_PER_WORKER // CHUNK
LANES = 16


def _encode(tok3, intensity_ids, seq_rep, int_pieces):
    mesh = plsc.VectorSubcoreMesh(core_axis_name="c", subcore_axis_name="s")

    @functools.partial(
        pl.kernel,
        mesh=mesh,
        out_type=jax.ShapeDtypeStruct((OUT_LEN, DBLK, BATCH, 128),
                                      jnp.float32),
        scratch_types=[
            pltpu.VMEM((TOKBLK, 128), jnp.int32),
            pltpu.VMEM((CHUNK, DBLK, 128), jnp.float32),
            pltpu.VMEM((CHUNK, DBLK, 128), jnp.float32),
            pltpu.VMEM((LANES,), jnp.int32),
            pltpu.VMEM((2 * LANES,), jnp.int32),
            pltpu.VMEM((2 * LANES, 128), jnp.float32),
            pltpu.SemaphoreType.DMA,
            pltpu.SemaphoreType.DMA,
            pltpu.SemaphoreType.DMA,
        ],
    )
    def k(tok_hbm, iid_hbm, seqt_hbm, intt_hbm, out_hbm,
          idx_v, buf0, buf1, iidx_v, ipidx_v, ibuf, sem0, sem1, isem):
        sid = lax.axis_index("s")
        wid = sid * 2 + lax.axis_index("c")
        b = wid // WORKERS_PER_BATCH
        part = wid % WORKERS_PER_BATCH
        dst_base = part * ROWS_PER_WORKER

        pltpu.sync_copy(tok_hbm.at[pl.ds(part * TOKBLK, TOKBLK), b, :],
                        idx_v)

        off = jnp.full((LANES,), SEQ_VOCAB, jnp.int32) * sid
        for r in range(TOKBLK):
            for c in range(128 // LANES):
                sl = pl.ds(c * LANES, LANES)
                idx_v[r, sl] = idx_v[r, sl] + off

        bufs = (buf0, buf1)
        sems = (sem0, sem1)

        def chunk_idx(j):
            return idx_v.at[j // 4, pl.ds((j % 4) * CHUNK, CHUNK)]

        copies = [None] * NCHUNKS
        copies[0] = pltpu.async_copy(seqt_hbm.at[chunk_idx(0)], buf0, sem0)
        for j in range(NCHUNKS):
            if j + 1 < NCHUNKS:
                copies[j + 1] = pltpu.async_copy(
                    seqt_hbm.at[chunk_idx(j + 1)],
                    bufs[(j + 1) % 2], sems[(j + 1) % 2])
            copies[j].wait()
            pltpu.sync_copy(
                bufs[j % 2],
                out_hbm.at[pl.ds(dst_base + j * CHUNK, CHUNK), :, b, :])

        @pl.when(wid == 0)
        def _():
            pltpu.sync_copy(iid_hbm, iidx_v.at[pl.ds(0, BATCH)])
            ids16 = iidx_v[pl.ds(0, LANES)]
            for v in range(2):
                j = lax.iota(jnp.int32, LANES) + (v * LANES)
                sel = j >> 3
                iv = j & (DBLK - 1)
                rid = lax.gather(
                    ids16, sel[:, None],
                    dimension_numbers=lax.GatherDimensionNumbers(
                        offset_dims=(), collapsed_slice_dims=(0,),
                        start_index_map=(0,)),
                    slice_sizes=(1,),
                    mode=lax.GatherScatterMode.PROMISE_IN_BOUNDS)
                pieces = (rid >> 3) * 64 + iv * 8 + (rid & 7)
                ipidx_v[pl.ds(v * LANES, LANES)] = pieces
            pltpu.async_copy(intt_hbm.at[ipidx_v], ibuf, isem).wait()
            for bb in range(BATCH):
                pltpu.sync_copy(ibuf.at[pl.ds(bb * DBLK, DBLK)],
                                out_hbm.at[SEQ_LEN, :, bb, :])

    return k(tok3, intensity_ids, seq_rep, int_pieces)


def kernel(sequence_tokens, intensity_ids, seq_table, int_table):
    tok3 = (sequence_tokens.astype(jnp.int32)
            .reshape(BATCH, SEQ_LEN // 128, 128).transpose(1, 0, 2))
    int_pieces = (int_table.reshape(8, DBLK, DBLK, 128)
                  .transpose(0, 2, 1, 3).reshape(64 * DBLK, 128))
    seq_rep = jnp.tile(seq_table, (NREP, 1)).reshape(
        NREP * SEQ_VOCAB, DBLK, 128)
    out4 = _encode(tok3, intensity_ids.astype(jnp.int32), seq_rep,
                   int_pieces)
    return out4.transpose(2, 0, 1, 3).reshape(BATCH, OUT_LEN, D_MODEL)


# speedup vs baseline: 3.1883x; 1.0705x over previous
import functools

import jax
import jax.numpy as jnp
from jax import lax
from jax.experimental import pallas as pl
from jax.experimental.pallas import tpu as pltpu
from jax.experimental.pallas import tpu_sc as plsc

D_MODEL = 1024
DBLK = D_MODEL // 128
BATCH = 4
SEQ_LEN = 2048
OUT_LEN = SEQ_LEN + 1
SEQ_VOCAB = 30
NUM_WORKERS = 32
NREP = 32
ROWS_PER_WORKER = BATCH * SEQ_LEN // NUM_WORKERS
WORKERS_PER_BATCH = NUM_WORKERS // BATCH
TOKBLK = ROWS_PER_WORKER // 128
CHUNK = 32
NCHUNKS = ROWS_PER_WORKER // CHUNK
LANES = 16


def _encode(tok3, intensity_ids, seq_rep, int_pieces):
    mesh = plsc.VectorSubcoreMesh(core_axis_name="c", subcore_axis_name="s")

    @functools.partial(
        pl.kernel,
        mesh=mesh,
        out_type=jax.ShapeDtypeStruct((OUT_LEN, DBLK, BATCH, 128),
                                      jnp.float32),
        scratch_types=[
            pltpu.VMEM((TOKBLK, 128), jnp.int32),
            pltpu.VMEM((CHUNK, DBLK, 128), jnp.float32),
            pltpu.VMEM((CHUNK, DBLK, 128), jnp.float32),
            pltpu.VMEM((LANES,), jnp.int32),
            pltpu.VMEM((2 * LANES,), jnp.int32),
            pltpu.VMEM((2 * LANES, 128), jnp.float32),
            pltpu.SemaphoreType.DMA,
            pltpu.SemaphoreType.DMA,
            pltpu.SemaphoreType.DMA,
        ],
    )
    def k(tok_hbm, iid_hbm, seqt_hbm, intt_hbm, out_hbm,
          idx_v, buf0, buf1, iidx_v, ipidx_v, ibuf, sem0, sem1, isem):
        sid = lax.axis_index("s")
        wid = sid * 2 + lax.axis_index("c")
        b = wid // WORKERS_PER_BATCH
        part = wid % WORKERS_PER_BATCH
        dst_base = part * ROWS_PER_WORKER

        pltpu.sync_copy(tok_hbm.at[pl.ds(part * TOKBLK, TOKBLK), b, :],
                        idx_v)

        off = jnp.full((LANES,), SEQ_VOCAB, jnp.int32) * wid
        for r in range(TOKBLK):
            for c in range(128 // LANES):
                sl = pl.ds(c * LANES, LANES)
                idx_v[r, sl] = idx_v[r, sl] + off

        bufs = (buf0, buf1)
        sems = (sem0, sem1)

        def chunk_idx(j):
            return idx_v.at[j // 4, pl.ds((j % 4) * CHUNK, CHUNK)]

        copies = [None] * NCHUNKS
        copies[0] = pltpu.async_copy(seqt_hbm.at[chunk_idx(0)], buf0, sem0)
        for j in range(NCHUNKS):
            if j + 1 < NCHUNKS:
                copies[j + 1] = pltpu.async_copy(
                    seqt_hbm.at[chunk_idx(j + 1)],
                    bufs[(j + 1) % 2], sems[(j + 1) % 2])
            copies[j].wait()
            pltpu.sync_copy(
                bufs[j % 2],
                out_hbm.at[pl.ds(dst_base + j * CHUNK, CHUNK), :, b, :])

        @pl.when(wid == 0)
        def _():
            pltpu.sync_copy(iid_hbm, iidx_v.at[pl.ds(0, BATCH)])
            ids16 = iidx_v[pl.ds(0, LANES)]
            for v in range(2):
                j = lax.iota(jnp.int32, LANES) + (v * LANES)
                sel = j >> 3
                iv = j & (DBLK - 1)
                rid = lax.gather(
                    ids16, sel[:, None],
                    dimension_numbers=lax.GatherDimensionNumbers(
                        offset_dims=(), collapsed_slice_dims=(0,),
                        start_index_map=(0,)),
                    slice_sizes=(1,),
                    mode=lax.GatherScatterMode.PROMISE_IN_BOUNDS)
                pieces = (rid >> 3) * 64 + iv * 8 + (rid & 7)
                ipidx_v[pl.ds(v * LANES, LANES)] = pieces
            pltpu.async_copy(intt_hbm.at[ipidx_v], ibuf, isem).wait()
            for bb in range(BATCH):
                pltpu.sync_copy(ibuf.at[pl.ds(bb * DBLK, DBLK)],
                                out_hbm.at[SEQ_LEN, :, bb, :])

    return k(tok3, intensity_ids, seq_rep, int_pieces)


def kernel(sequence_tokens, intensity_ids, seq_table, int_table):
    tok3 = (sequence_tokens.astype(jnp.int32)
            .reshape(BATCH, SEQ_LEN // 128, 128).transpose(1, 0, 2))
    int_pieces = (int_table.reshape(8, DBLK, DBLK, 128)
                  .transpose(0, 2, 1, 3).reshape(64 * DBLK, 128))
    seq_rep = jnp.tile(seq_table, (NREP, 1)).reshape(
        NREP * SEQ_VOCAB, DBLK, 128)
    out4 = _encode(tok3, intensity_ids.astype(jnp.int32), seq_rep,
                   int_pieces)
    return out4.transpose(2, 0, 1, 3).reshape(BATCH, OUT_LEN, D_MODEL)
